# baseline (device time: 47656 ns/iter reference)
import jax
import jax.numpy as jnp
from jax import lax
from jax.experimental import pallas as pl
from jax.experimental.pallas import tpu as pltpu

N_DEV = 8
BLK = 64


def kernel(x, Wq, K_ext, V_ext, Wo):
    B, Sq, D = x.shape
    Hq, Dh = K_ext.shape[2], K_ext.shape[3]
    HD = Hq * Dh
    Dout = Wo.shape[1]

    def body(x_ref, wq_ref, k_ref, v_ref, wo_ref, out_ref,
             comm_ref, send_sems, recv_sems):
        my_i = lax.axis_index("i")
        left = lax.rem(my_i + (N_DEV - 1), N_DEV)
        right = lax.rem(my_i + 1, N_DEV)

        barrier_sem = pltpu.get_barrier_semaphore()
        pl.semaphore_signal(barrier_sem, inc=1, device_id=(left,),
                            device_id_type=pl.DeviceIdType.MESH)
        pl.semaphore_signal(barrier_sem, inc=1, device_id=(right,),
                            device_id_type=pl.DeviceIdType.MESH)
        pl.semaphore_wait(barrier_sem, 2)

        qb = lax.broadcasted_iota(jnp.int32, (Sq, Sq), 0) // BLK
        kb = lax.broadcasted_iota(jnp.int32, (Sq, Sq), 1) // BLK
        mask = (qb == kb) | (kb == 0) | (((qb + kb) % 3) == 0)

        col0 = my_i * HD
        wq_loc = wq_ref[:, pl.ds(col0, HD)]
        for b in range(B):
            xb = x_ref[b]
            q_all = jnp.dot(xb, wq_loc,
                            preferred_element_type=jnp.float32)
            for h in range(Hq):
                q_h = q_all[:, h * Dh:(h + 1) * Dh]
                k_h = k_ref[b, :, h, :]
                v_h = v_ref[b, :, h, :]
                s = lax.dot_general(q_h, k_h, (((1,), (1,)), ((), ())),
                                    preferred_element_type=jnp.float32)
                s = s * 0.125
                s = jnp.where(mask, s, -1e9)
                m = jnp.max(s, axis=1, keepdims=True)
                w = jnp.exp(s - m)
                w = w / jnp.sum(w, axis=1, keepdims=True)
                ctx = jnp.dot(w, v_h,
                              preferred_element_type=jnp.float32)
                comm_ref[0, b, :, h * Dh:(h + 1) * Dh] = ctx

        wo_me = wo_ref[pl.ds(col0, HD), :]
        for b in range(B):
            out_ref[b] = jnp.dot(comm_ref[0, b], wo_me,
                                 preferred_element_type=jnp.float32)

        for h in range(N_DEV - 1):
            rdma = pltpu.make_async_remote_copy(
                src_ref=comm_ref.at[h],
                dst_ref=comm_ref.at[h + 1],
                send_sem=send_sems.at[h],
                recv_sem=recv_sems.at[h],
                device_id=(right,),
                device_id_type=pl.DeviceIdType.MESH,
            )
            rdma.start()
            rdma.wait()
            origin = lax.rem(my_i - (h + 1) + N_DEV, N_DEV)
            wo_s = wo_ref[pl.ds(origin * HD, HD), :]
            for b in range(B):
                out_ref[b] += jnp.dot(comm_ref[h + 1, b], wo_s,
                                      preferred_element_type=jnp.float32)

    return pl.pallas_call(
        body,
        out_shape=jax.ShapeDtypeStruct((B, Sq, Dout), jnp.float32),
        in_specs=[pl.BlockSpec(memory_space=pltpu.VMEM)] * 5,
        out_specs=pl.BlockSpec(memory_space=pltpu.VMEM),
        scratch_shapes=[
            pltpu.VMEM((N_DEV, B, Sq, HD), jnp.float32),
            pltpu.SemaphoreType.DMA((N_DEV - 1,)),
            pltpu.SemaphoreType.DMA((N_DEV - 1,)),
        ],
        compiler_params=pltpu.CompilerParams(collective_id=0),
    )(x, Wq, K_ext, V_ext, Wo)


# device time: 24593 ns/iter; 1.9378x vs baseline; 1.9378x over previous
import jax
import jax.numpy as jnp
from jax import lax
from jax.experimental import pallas as pl
from jax.experimental.pallas import tpu as pltpu

N_DEV = 8
BLK = 64


def kernel(x, Wq, K_ext, V_ext, Wo):
    B, Sq, D = x.shape
    Hq, Dh = K_ext.shape[2], K_ext.shape[3]
    HD = Hq * Dh
    Dout = Wo.shape[1]

    def body(x_ref, wq_ref, k_ref, v_ref, wo_ref, out_ref,
             comm_ref, send_sems, recv_sems):
        my_i = lax.axis_index("i")

        barrier_sem = pltpu.get_barrier_semaphore()
        for j in range(1, N_DEV):
            peer = lax.rem(my_i + j, N_DEV)
            pl.semaphore_signal(barrier_sem, inc=1, device_id=(peer,),
                                device_id_type=pl.DeviceIdType.MESH)
        pl.semaphore_wait(barrier_sem, N_DEV - 1)

        qb = lax.broadcasted_iota(jnp.int32, (Sq, Sq), 0) // BLK
        kb = lax.broadcasted_iota(jnp.int32, (Sq, Sq), 1) // BLK
        mask = (qb == kb) | (kb == 0) | (((qb + kb) % 3) == 0)

        col0 = my_i * HD
        wq_loc = wq_ref[:, pl.ds(col0, HD)]
        for b in range(B):
            xb = x_ref[b]
            q_all = jnp.dot(xb, wq_loc,
                            preferred_element_type=jnp.float32)
            for h in range(Hq):
                q_h = q_all[:, h * Dh:(h + 1) * Dh]
                k_h = k_ref[b, :, h, :]
                v_h = v_ref[b, :, h, :]
                s = lax.dot_general(q_h, k_h, (((1,), (1,)), ((), ())),
                                    preferred_element_type=jnp.float32)
                s = s * 0.125
                s = jnp.where(mask, s, -1e9)
                m = jnp.max(s, axis=1, keepdims=True)
                w = jnp.exp(s - m)
                w = w / jnp.sum(w, axis=1, keepdims=True)
                ctx = jnp.dot(w, v_h,
                              preferred_element_type=jnp.float32)
                comm_ref[my_i, b, :, h * Dh:(h + 1) * Dh] = (
                    ctx.astype(jnp.bfloat16))

        sends = []
        for j in range(1, N_DEV):
            peer = lax.rem(my_i + j, N_DEV)
            rdma = pltpu.make_async_remote_copy(
                src_ref=comm_ref.at[my_i],
                dst_ref=comm_ref.at[my_i],
                send_sem=send_sems.at[j - 1],
                recv_sem=recv_sems.at[my_i],
                device_id=(peer,),
                device_id_type=pl.DeviceIdType.MESH,
            )
            rdma.start()
            sends.append(rdma)

        wo_me = wo_ref[pl.ds(col0, HD), :]
        for b in range(B):
            own = comm_ref[my_i, b].astype(jnp.float32)
            out_ref[b] = jnp.dot(own, wo_me,
                                 preferred_element_type=jnp.float32)

        for j in range(1, N_DEV):
            origin = lax.rem(my_i + j, N_DEV)
            recv = pltpu.make_async_remote_copy(
                src_ref=comm_ref.at[origin],
                dst_ref=comm_ref.at[origin],
                send_sem=send_sems.at[j - 1],
                recv_sem=recv_sems.at[origin],
                device_id=(origin,),
                device_id_type=pl.DeviceIdType.MESH,
            )
            recv.wait_recv()
            wo_s = wo_ref[pl.ds(origin * HD, HD), :]
            for b in range(B):
                chunk = comm_ref[origin, b].astype(jnp.float32)
                out_ref[b] += jnp.dot(chunk, wo_s,
                                      preferred_element_type=jnp.float32)

        for rdma in sends:
            rdma.wait_send()

    return pl.pallas_call(
        body,
        out_shape=jax.ShapeDtypeStruct((B, Sq, Dout), jnp.float32),
        in_specs=[pl.BlockSpec(memory_space=pltpu.VMEM)] * 5,
        out_specs=pl.BlockSpec(memory_space=pltpu.VMEM),
        scratch_shapes=[
            pltpu.VMEM((N_DEV, B, Sq, HD), jnp.bfloat16),
            pltpu.SemaphoreType.DMA((N_DEV - 1,)),
            pltpu.SemaphoreType.DMA((N_DEV,)),
        ],
        compiler_params=pltpu.CompilerParams(collective_id=0),
    )(x, Wq, K_ext, V_ext, Wo)
